# baseline (speedup 1.0000x reference)
import functools

import numpy as np
import jax
import jax.numpy as jnp
from jax import lax
from jax.experimental import pallas as pl
from jax.experimental.pallas import tpu as pltpu
from jax.experimental.pallas import tpu_sc as plsc

_TABLE_NUM = 16
_TABLE_SIZE = 8192
_IMG = 256
_N = _IMG * _IMG
_BATCH = 4
_HID = 64


def _resolutions():
    b = np.exp((np.log(256.0) - np.log(16.0)) / (_TABLE_NUM - 1))
    return np.floor(16.0 * (b ** np.arange(_TABLE_NUM))).astype(np.int64)


def _build_consts():
    res = _resolutions()
    pi2 = np.uint32(2654435761)
    c = (np.arange(_IMG, dtype=np.float32) + np.float32(0.5)) / np.float32(_IMG)
    hA = np.empty((_TABLE_NUM, _N), np.uint32)
    hB = np.empty((_TABLE_NUM, _N), np.uint32)
    ws = np.empty((4, _TABLE_NUM, _N), np.float32)
    for lvl in range(_TABLE_NUM):
        r = np.float32(float(res[lvl]))
        xy = c * r
        x0 = np.floor(xy)
        fr = (xy - x0).astype(np.float32)
        i0 = x0.astype(np.uint32)
        i1 = i0 + np.uint32(1)
        ix0 = i0[:, None]
        ix1 = i1[:, None]
        m0 = (i0 * pi2)[None, :]
        m1 = (i1 * pi2)[None, :]

        def hsh(a, m):
            return (a ^ m) % np.uint32(_TABLE_SIZE)

        hA[lvl] = (hsh(ix0, m0) | (hsh(ix0, m1) << np.uint32(16))).ravel()
        hB[lvl] = (hsh(ix1, m0) | (hsh(ix1, m1) << np.uint32(16))).ravel()
        fx = np.broadcast_to(fr[:, None], (_IMG, _IMG)).astype(np.float32)
        fy = np.broadcast_to(fr[None, :], (_IMG, _IMG)).astype(np.float32)
        gx = np.float32(1.0) - fx
        gy = np.float32(1.0) - fy
        ws[0, lvl] = (gx * gy).ravel()
        ws[1, lvl] = (gx * fy).ravel()
        ws[2, lvl] = (fx * gy).ravel()
        ws[3, lvl] = (fx * fy).ravel()
    return hA.view(np.int32), hB.view(np.int32), ws


_HA, _HB, _WS = _build_consts()
_W00, _W01, _W10, _W11 = _WS[0], _WS[1], _WS[2], _WS[3]


_CB = 8192


def _tablegen_body(z_ref, wg_ref, bg_ref, out_ref):
    acc = jnp.dot(z_ref[...], wg_ref[...], preferred_element_type=jnp.float32)
    out_ref[...] = jnp.tanh(acc + bg_ref[...]).astype(jnp.bfloat16)


def _tablegen(z, Wg, bg):
    ncols = Wg.shape[1]
    return pl.pallas_call(
        _tablegen_body,
        grid=(ncols // _CB,),
        in_specs=[
            pl.BlockSpec((_BATCH, 64), lambda i: (0, 0)),
            pl.BlockSpec((64, _CB), lambda i: (0, i)),
            pl.BlockSpec((1, _CB), lambda i: (0, i)),
        ],
        out_specs=pl.BlockSpec((_BATCH, _CB), lambda i: (0, i)),
        out_shape=jax.ShapeDtypeStruct((_BATCH, ncols), jnp.bfloat16),
    )(z, Wg, bg.reshape(1, -1))


_NC = 2
_CHUNK = 4096
_M16 = np.int32(0xFFFF)
_MHI = np.int32(-65536)


def _unpack_pair(g):
    lo = plsc.bitcast(lax.shift_left(g, 16), jnp.float32)
    hi = plsc.bitcast(jnp.bitwise_and(g, _MHI), jnp.float32)
    return lo, hi


@functools.cache
def _make_sc_gather():
    return functools.partial(
        pl.kernel,
        mesh=plsc.VectorSubcoreMesh(core_axis_name="c", subcore_axis_name="s"),
        compiler_params=pltpu.CompilerParams(needs_layout_passes=False),
        out_type=jax.ShapeDtypeStruct((_BATCH, 2 * _TABLE_NUM, _N), jnp.float32),
        scratch_types=[
            pltpu.VMEM((_TABLE_SIZE,), jnp.int32),
            pltpu.VMEM((_TABLE_SIZE,), jnp.int32),
            pltpu.VMEM((_CHUNK,), jnp.int32),
            pltpu.VMEM((_CHUNK,), jnp.int32),
            pltpu.VMEM((_CHUNK,), jnp.float32),
            pltpu.VMEM((_CHUNK,), jnp.float32),
            pltpu.VMEM((_CHUNK,), jnp.float32),
            pltpu.VMEM((_CHUNK,), jnp.float32),
            pltpu.VMEM((2, _CHUNK), jnp.float32),
            pltpu.VMEM((2, _CHUNK), jnp.float32),
        ],
    )(_sc_gather_body)


def _sc_gather_body(tables, hA, hB, w00, w01, w10, w11, featT,
                    tbl0_v, tbl1_v, hA_v, hB_v, w00_v, w01_v, w10_v, w11_v,
                    st0_v, st1_v):
    wid = lax.axis_index("s") * _NC + lax.axis_index("c")
    lvl = wid % _TABLE_NUM
    b0 = (wid // _TABLE_NUM) * 2
    pltpu.sync_copy(tables.at[b0, lvl], tbl0_v)
    pltpu.sync_copy(tables.at[b0 + 1, lvl], tbl1_v)

    def chunk_body(ci, carry):
        off = ci * _CHUNK
        pltpu.sync_copy(hA.at[lvl, pl.ds(off, _CHUNK)], hA_v)
        pltpu.sync_copy(hB.at[lvl, pl.ds(off, _CHUNK)], hB_v)
        pltpu.sync_copy(w00.at[lvl, pl.ds(off, _CHUNK)], w00_v)
        pltpu.sync_copy(w01.at[lvl, pl.ds(off, _CHUNK)], w01_v)
        pltpu.sync_copy(w10.at[lvl, pl.ds(off, _CHUNK)], w10_v)
        pltpu.sync_copy(w11.at[lvl, pl.ds(off, _CHUNK)], w11_v)

        def grp(g, c2):
            s = g * 16
            va = hA_v[pl.ds(s, 16)]
            vb = hB_v[pl.ds(s, 16)]
            i00 = jnp.bitwise_and(va, _M16)
            i01 = lax.shift_right_logical(va, 16)
            i10 = jnp.bitwise_and(vb, _M16)
            i11 = lax.shift_right_logical(vb, 16)
            w00v = w00_v[pl.ds(s, 16)]
            w01v = w01_v[pl.ds(s, 16)]
            w10v = w10_v[pl.ds(s, 16)]
            w11v = w11_v[pl.ds(s, 16)]
            for tbl_v, st_v in ((tbl0_v, st0_v), (tbl1_v, st1_v)):
                f00a, f00b = _unpack_pair(plsc.load_gather(tbl_v, [i00]))
                f01a, f01b = _unpack_pair(plsc.load_gather(tbl_v, [i01]))
                f10a, f10b = _unpack_pair(plsc.load_gather(tbl_v, [i10]))
                f11a, f11b = _unpack_pair(plsc.load_gather(tbl_v, [i11]))
                ra = ((f00a * w00v + f01a * w01v) + f10a * w10v) + f11a * w11v
                rb = ((f00b * w00v + f01b * w01v) + f10b * w10v) + f11b * w11v
                st_v[0, pl.ds(s, 16)] = ra
                st_v[1, pl.ds(s, 16)] = rb
            return c2

        lax.fori_loop(0, _CHUNK // 16, grp, 0)
        pltpu.sync_copy(st0_v, featT.at[b0, pl.ds(lvl * 2, 2), pl.ds(off, _CHUNK)])
        pltpu.sync_copy(st1_v, featT.at[b0 + 1, pl.ds(lvl * 2, 2), pl.ds(off, _CHUNK)])
        return carry

    lax.fori_loop(0, _N // _CHUNK, chunk_body, 0)


_TN = 2048


def _mlp_body(x_ref, w1_ref, b1_ref, w2_ref, b2_ref, w3_ref, b3_ref, out_ref):
    x = x_ref[0]
    h = jnp.dot(w1_ref[...], x, preferred_element_type=jnp.float32)
    h = jnp.maximum(h + b1_ref[...], 0.0)
    h = jnp.dot(w2_ref[...], h, preferred_element_type=jnp.float32)
    h = jnp.maximum(h + b2_ref[...], 0.0)
    o = jnp.dot(w3_ref[...], h, preferred_element_type=jnp.float32)
    out_ref[0] = jnp.tanh(o + b3_ref[...])


def _mlp(featT, W1T, b1, W2T, b2, W3T, b3):
    return pl.pallas_call(
        _mlp_body,
        grid=(_BATCH, _N // _TN),
        in_specs=[
            pl.BlockSpec((1, 2 * _TABLE_NUM, _TN), lambda b, i: (b, 0, i)),
            pl.BlockSpec((_HID, 2 * _TABLE_NUM), lambda b, i: (0, 0)),
            pl.BlockSpec((_HID, 1), lambda b, i: (0, 0)),
            pl.BlockSpec((_HID, _HID), lambda b, i: (0, 0)),
            pl.BlockSpec((_HID, 1), lambda b, i: (0, 0)),
            pl.BlockSpec((3, _HID), lambda b, i: (0, 0)),
            pl.BlockSpec((3, 1), lambda b, i: (0, 0)),
        ],
        out_specs=pl.BlockSpec((1, 3, _TN), lambda b, i: (b, 0, i)),
        out_shape=jax.ShapeDtypeStruct((_BATCH, 3, _N), jnp.float32),
    )(featT, W1T, b1.reshape(-1, 1), W2T, b2.reshape(-1, 1),
      W3T, b3.reshape(-1, 1))


def kernel(z, Wg, bg, W1, b1, W2, b2, W3, b3):
    tables_bf = _tablegen(z, Wg, bg)
    tables = lax.bitcast_convert_type(
        tables_bf.reshape(_BATCH, _TABLE_NUM, _TABLE_SIZE, 2), jnp.int32)
    featT = _make_sc_gather()(tables, _HA, _HB, _W00, _W01, _W10, _W11)
    out = _mlp(featT, W1.T, b1, W2.T, b2, W3.T, b3)
    return out.reshape(_BATCH, 3, _IMG, _IMG)


# speedup vs baseline: 1.8221x; 1.8221x over previous
import functools

import numpy as np
import jax
import jax.numpy as jnp
from jax import lax
from jax.experimental import pallas as pl
from jax.experimental.pallas import tpu as pltpu
from jax.experimental.pallas import tpu_sc as plsc

_TABLE_NUM = 16
_TABLE_SIZE = 8192
_IMG = 256
_N = _IMG * _IMG
_BATCH = 4
_HID = 64
_CHUNK = 2048
_NCHUNK = _N // _CHUNK
_M16 = np.int32(0xFFFF)
_MHI = np.int32(-65536)
_RND = np.int32(0x7FFF)


def _resolutions():
    b = np.exp((np.log(256.0) - np.log(16.0)) / (_TABLE_NUM - 1))
    return np.floor(16.0 * (b ** np.arange(_TABLE_NUM))).astype(np.int64)


def _build_consts():
    res = _resolutions()
    pi2 = np.uint32(2654435761)
    c = (np.arange(_IMG, dtype=np.float32) + np.float32(0.5)) / np.float32(_IMG)
    cc = np.empty((_TABLE_NUM, 6, _N), np.float32)
    for lvl in range(_TABLE_NUM):
        r = np.float32(float(res[lvl]))
        xy = c * r
        x0 = np.floor(xy)
        fr = (xy - x0).astype(np.float32)
        i0 = x0.astype(np.uint32)
        i1 = i0 + np.uint32(1)
        ix0 = i0[:, None]
        ix1 = i1[:, None]
        m0 = (i0 * pi2)[None, :]
        m1 = (i1 * pi2)[None, :]

        def hsh(a, m):
            return (a ^ m) % np.uint32(_TABLE_SIZE)

        hA = (hsh(ix0, m0) | (hsh(ix0, m1) << np.uint32(16))).ravel()
        hB = (hsh(ix1, m0) | (hsh(ix1, m1) << np.uint32(16))).ravel()
        cc[lvl, 0] = hA.view(np.float32)
        cc[lvl, 1] = hB.view(np.float32)
        fx = np.broadcast_to(fr[:, None], (_IMG, _IMG)).astype(np.float32)
        fy = np.broadcast_to(fr[None, :], (_IMG, _IMG)).astype(np.float32)
        gx = np.float32(1.0) - fx
        gy = np.float32(1.0) - fy
        cc[lvl, 2] = (gx * gy).ravel()
        cc[lvl, 3] = (gx * fy).ravel()
        cc[lvl, 4] = (fx * gy).ravel()
        cc[lvl, 5] = (fx * fy).ravel()
    return np.ascontiguousarray(
        cc.reshape(_TABLE_NUM, 6, _NCHUNK, _CHUNK).transpose(0, 2, 1, 3))


_CC = _build_consts()


_CB = 8192


def _tablegen_body(z_ref, wg_ref, bg_ref, out_ref):
    acc = jnp.dot(z_ref[...], wg_ref[...], preferred_element_type=jnp.float32)
    out_ref[...] = jnp.tanh(acc + bg_ref[...])


def _tablegen(z, Wg, bg):
    ncols = Wg.shape[1]
    return pl.pallas_call(
        _tablegen_body,
        grid=(ncols // _CB,),
        in_specs=[
            pl.BlockSpec((_BATCH, 64), lambda i: (0, 0)),
            pl.BlockSpec((64, _CB), lambda i: (0, i)),
            pl.BlockSpec((1, _CB), lambda i: (0, i)),
        ],
        out_specs=pl.BlockSpec((_BATCH, _CB), lambda i: (0, i)),
        out_shape=jax.ShapeDtypeStruct((_BATCH, ncols), jnp.float32),
    )(z, Wg, bg.reshape(1, -1))


_NC = 2


def _pack_table(tblf_v, pk_v):
    iota2 = lax.iota(jnp.int32, 16) * 2

    def body(g, carry):
        ii = iota2 + g * 32
        be = plsc.bitcast(plsc.load_gather(tblf_v, [ii]), jnp.int32)
        bo = plsc.bitcast(plsc.load_gather(tblf_v, [ii + 1]), jnp.int32)
        se = be + jnp.bitwise_and(lax.shift_right_logical(be, 16), 1) + _RND
        so = bo + jnp.bitwise_and(lax.shift_right_logical(bo, 16), 1) + _RND
        word = jnp.bitwise_or(lax.shift_right_logical(se, 16),
                              jnp.bitwise_and(so, _MHI))
        pk_v[pl.ds(g * 16, 16)] = word
        return carry

    lax.fori_loop(0, _TABLE_SIZE // 16, body, 0)


def _unpack_pair(g):
    lo = plsc.bitcast(lax.shift_left(g, 16), jnp.float32)
    hi = plsc.bitcast(jnp.bitwise_and(g, _MHI), jnp.float32)
    return lo, hi


@functools.cache
def _make_sc_gather():
    return functools.partial(
        pl.kernel,
        mesh=plsc.VectorSubcoreMesh(core_axis_name="c", subcore_axis_name="s"),
        compiler_params=pltpu.CompilerParams(needs_layout_passes=False),
        out_type=jax.ShapeDtypeStruct((_BATCH, 2 * _TABLE_NUM, _N), jnp.float32),
        scratch_types=[
            pltpu.VMEM((2 * _TABLE_SIZE,), jnp.float32),
            pltpu.VMEM((2 * _TABLE_SIZE,), jnp.float32),
            pltpu.VMEM((_TABLE_SIZE,), jnp.int32),
            pltpu.VMEM((_TABLE_SIZE,), jnp.int32),
            pltpu.VMEM((6, _CHUNK), jnp.float32),
            pltpu.VMEM((2, _CHUNK), jnp.float32),
            pltpu.VMEM((2, _CHUNK), jnp.float32),
        ],
    )(_sc_gather_body)


def _sc_gather_body(tables, cc, featT,
                    tf0_v, tf1_v, tbl0_v, tbl1_v, cb_v, st0_v, st1_v):
    wid = lax.axis_index("s") * _NC + lax.axis_index("c")
    lvl = wid % _TABLE_NUM
    b0 = (wid // _TABLE_NUM) * 2
    pltpu.sync_copy(tables.at[b0, lvl], tf0_v)
    pltpu.sync_copy(tables.at[b0 + 1, lvl], tf1_v)
    _pack_table(tf0_v, tbl0_v)
    _pack_table(tf1_v, tbl1_v)

    def chunk_body(ci, carry):
        off = ci * _CHUNK
        pltpu.sync_copy(cc.at[lvl, ci], cb_v)

        def grp(g, c2):
            s = g * 16
            va = plsc.bitcast(cb_v[0, pl.ds(s, 16)], jnp.int32)
            vb = plsc.bitcast(cb_v[1, pl.ds(s, 16)], jnp.int32)
            i00 = jnp.bitwise_and(va, _M16)
            i01 = lax.shift_right_logical(va, 16)
            i10 = jnp.bitwise_and(vb, _M16)
            i11 = lax.shift_right_logical(vb, 16)
            w00v = cb_v[2, pl.ds(s, 16)]
            w01v = cb_v[3, pl.ds(s, 16)]
            w10v = cb_v[4, pl.ds(s, 16)]
            w11v = cb_v[5, pl.ds(s, 16)]
            for tbl_v, st_v in ((tbl0_v, st0_v), (tbl1_v, st1_v)):
                f00a, f00b = _unpack_pair(plsc.load_gather(tbl_v, [i00]))
                f01a, f01b = _unpack_pair(plsc.load_gather(tbl_v, [i01]))
                f10a, f10b = _unpack_pair(plsc.load_gather(tbl_v, [i10]))
                f11a, f11b = _unpack_pair(plsc.load_gather(tbl_v, [i11]))
                ra = ((f00a * w00v + f01a * w01v) + f10a * w10v) + f11a * w11v
                rb = ((f00b * w00v + f01b * w01v) + f10b * w10v) + f11b * w11v
                st_v[0, pl.ds(s, 16)] = ra
                st_v[1, pl.ds(s, 16)] = rb
            return c2

        lax.fori_loop(0, _CHUNK // 16, grp, 0)
        pltpu.sync_copy(st0_v, featT.at[b0, pl.ds(lvl * 2, 2), pl.ds(off, _CHUNK)])
        pltpu.sync_copy(st1_v, featT.at[b0 + 1, pl.ds(lvl * 2, 2), pl.ds(off, _CHUNK)])
        return carry

    lax.fori_loop(0, _NCHUNK, chunk_body, 0)


_TN = 2048


def _mlp_body(x_ref, w1_ref, b1_ref, w2_ref, b2_ref, w3_ref, b3_ref, out_ref):
    x = x_ref[0]
    h = jnp.dot(w1_ref[...], x, preferred_element_type=jnp.float32)
    h = jnp.maximum(h + b1_ref[...], 0.0)
    h = jnp.dot(w2_ref[...], h, preferred_element_type=jnp.float32)
    h = jnp.maximum(h + b2_ref[...], 0.0)
    o = jnp.dot(w3_ref[...], h, preferred_element_type=jnp.float32)
    out_ref[0] = jnp.tanh(o + b3_ref[...])


def _mlp(featT, W1T, b1, W2T, b2, W3T, b3):
    return pl.pallas_call(
        _mlp_body,
        grid=(_BATCH, _N // _TN),
        in_specs=[
            pl.BlockSpec((1, 2 * _TABLE_NUM, _TN), lambda b, i: (b, 0, i)),
            pl.BlockSpec((_HID, 2 * _TABLE_NUM), lambda b, i: (0, 0)),
            pl.BlockSpec((_HID, 1), lambda b, i: (0, 0)),
            pl.BlockSpec((_HID, _HID), lambda b, i: (0, 0)),
            pl.BlockSpec((_HID, 1), lambda b, i: (0, 0)),
            pl.BlockSpec((3, _HID), lambda b, i: (0, 0)),
            pl.BlockSpec((3, 1), lambda b, i: (0, 0)),
        ],
        out_specs=pl.BlockSpec((1, 3, _TN), lambda b, i: (b, 0, i)),
        out_shape=jax.ShapeDtypeStruct((_BATCH, 3, _N), jnp.float32),
    )(featT, W1T, b1.reshape(-1, 1), W2T, b2.reshape(-1, 1),
      W3T, b3.reshape(-1, 1))


def kernel(z, Wg, bg, W1, b1, W2, b2, W3, b3):
    tables = _tablegen(z, Wg, bg).reshape(
        _BATCH, _TABLE_NUM, 2 * _TABLE_SIZE)
    featT = _make_sc_gather()(tables, _CC)
    out = _mlp(featT, W1.T, b1, W2.T, b2, W3.T, b3)
    return out.reshape(_BATCH, 3, _IMG, _IMG)


# speedup vs baseline: 2.2004x; 1.2076x over previous
import functools

import numpy as np
import jax
import jax.numpy as jnp
from jax import lax
from jax.experimental import pallas as pl
from jax.experimental.pallas import tpu as pltpu
from jax.experimental.pallas import tpu_sc as plsc

_TABLE_NUM = 16
_TABLE_SIZE = 8192
_IMG = 256
_N = _IMG * _IMG
_BATCH = 4
_HID = 64
_CHUNK = 2048
_NCHUNK = _N // _CHUNK
_M16 = np.int32(0xFFFF)
_MHI = np.int32(-65536)
_RND = np.int32(0x7FFF)


def _resolutions():
    b = np.exp((np.log(256.0) - np.log(16.0)) / (_TABLE_NUM - 1))
    return np.floor(16.0 * (b ** np.arange(_TABLE_NUM))).astype(np.int64)


def _build_consts():
    res = _resolutions()
    pi2 = np.uint32(2654435761)
    c = (np.arange(_IMG, dtype=np.float32) + np.float32(0.5)) / np.float32(_IMG)
    cc = np.empty((_TABLE_NUM, 6, _N), np.float32)
    for lvl in range(_TABLE_NUM):
        r = np.float32(float(res[lvl]))
        xy = c * r
        x0 = np.floor(xy)
        fr = (xy - x0).astype(np.float32)
        i0 = x0.astype(np.uint32)
        i1 = i0 + np.uint32(1)
        ix0 = i0[:, None]
        ix1 = i1[:, None]
        m0 = (i0 * pi2)[None, :]
        m1 = (i1 * pi2)[None, :]

        def hsh(a, m):
            return (a ^ m) % np.uint32(_TABLE_SIZE)

        hA = (hsh(ix0, m0) | (hsh(ix0, m1) << np.uint32(16))).ravel()
        hB = (hsh(ix1, m0) | (hsh(ix1, m1) << np.uint32(16))).ravel()
        cc[lvl, 0] = hA.view(np.float32)
        cc[lvl, 1] = hB.view(np.float32)
        fx = np.broadcast_to(fr[:, None], (_IMG, _IMG)).astype(np.float32)
        fy = np.broadcast_to(fr[None, :], (_IMG, _IMG)).astype(np.float32)
        gx = np.float32(1.0) - fx
        gy = np.float32(1.0) - fy
        cc[lvl, 2] = (gx * gy).ravel()
        cc[lvl, 3] = (gx * fy).ravel()
        cc[lvl, 4] = (fx * gy).ravel()
        cc[lvl, 5] = (fx * fy).ravel()
    return np.ascontiguousarray(
        cc.reshape(_TABLE_NUM, 6, _NCHUNK, _CHUNK).transpose(0, 2, 1, 3))


_CC = _build_consts()


_CB = 32768


def _tablegen_body(z_ref, wg_ref, bg_ref, out_ref):
    acc = jnp.dot(z_ref[...], wg_ref[...], preferred_element_type=jnp.float32)
    out_ref[...] = jnp.tanh(acc + bg_ref[...])


def _tablegen(z, Wg, bg):
    ncols = Wg.shape[1]
    return pl.pallas_call(
        _tablegen_body,
        grid=(ncols // _CB,),
        in_specs=[
            pl.BlockSpec((_BATCH, 64), lambda i: (0, 0)),
            pl.BlockSpec((64, _CB), lambda i: (0, i)),
            pl.BlockSpec((1, _CB), lambda i: (0, i)),
        ],
        out_specs=pl.BlockSpec((_BATCH, _CB), lambda i: (0, i)),
        out_shape=jax.ShapeDtypeStruct((_BATCH, ncols), jnp.float32),
    )(z, Wg, bg.reshape(1, -1))


_NC = 2


def _pack_table(tblf_v, pk_v):
    iota2 = lax.iota(jnp.int32, 16) * 2

    def body(g, carry):
        ii = iota2 + g * 32
        be = plsc.bitcast(plsc.load_gather(tblf_v, [ii]), jnp.int32)
        bo = plsc.bitcast(plsc.load_gather(tblf_v, [ii + 1]), jnp.int32)
        se = be + jnp.bitwise_and(lax.shift_right_logical(be, 16), 1) + _RND
        so = bo + jnp.bitwise_and(lax.shift_right_logical(bo, 16), 1) + _RND
        word = jnp.bitwise_or(lax.shift_right_logical(se, 16),
                              jnp.bitwise_and(so, _MHI))
        pk_v[pl.ds(g * 16, 16)] = word
        return carry

    lax.fori_loop(0, _TABLE_SIZE // 16, body, 0)


def _unpack_pair(g):
    lo = plsc.bitcast(lax.shift_left(g, 16), jnp.float32)
    hi = plsc.bitcast(jnp.bitwise_and(g, _MHI), jnp.float32)
    return lo, hi


@functools.cache
def _make_sc_gather():
    return functools.partial(
        pl.kernel,
        mesh=plsc.VectorSubcoreMesh(core_axis_name="c", subcore_axis_name="s"),
        compiler_params=pltpu.CompilerParams(needs_layout_passes=False),
        out_type=jax.ShapeDtypeStruct((_BATCH, 2 * _TABLE_NUM, _N), jnp.float32),
        scratch_types=[
            pltpu.VMEM((2 * _TABLE_SIZE,), jnp.float32),
            pltpu.VMEM((2 * _TABLE_SIZE,), jnp.float32),
            pltpu.VMEM((_TABLE_SIZE,), jnp.int32),
            pltpu.VMEM((_TABLE_SIZE,), jnp.int32),
            pltpu.VMEM((2, 6, _CHUNK), jnp.float32),
            pltpu.VMEM((2, 2, _CHUNK), jnp.float32),
            pltpu.VMEM((2, 2, _CHUNK), jnp.float32),
            pltpu.SemaphoreType.DMA,
            pltpu.SemaphoreType.DMA,
            pltpu.SemaphoreType.DMA,
            pltpu.SemaphoreType.DMA,
            pltpu.SemaphoreType.DMA,
            pltpu.SemaphoreType.DMA,
        ],
    )(_sc_gather_body)


def _sc_gather_body(tables, cc, featT,
                    tf0_v, tf1_v, tbl0_v, tbl1_v, cb_v, st0_v, st1_v,
                    sin0, sin1, so0a, so0b, so1a, so1b):
    wid = lax.axis_index("s") * _NC + lax.axis_index("c")
    lvl = wid % _TABLE_NUM
    b0 = (wid // _TABLE_NUM) * 2
    pltpu.sync_copy(tables.at[b0, lvl], tf0_v)
    pltpu.sync_copy(tables.at[b0 + 1, lvl], tf1_v)
    _pack_table(tf0_v, tbl0_v)
    _pack_table(tf1_v, tbl1_v)

    sin = (sin0, sin1)
    sout = ((so0a, so0b), (so1a, so1b))

    def compute_chunk(p, ci):
        def grp(g, c2):
            s = g * 16
            va = plsc.bitcast(cb_v[p, 0, pl.ds(s, 16)], jnp.int32)
            vb = plsc.bitcast(cb_v[p, 1, pl.ds(s, 16)], jnp.int32)
            i00 = jnp.bitwise_and(va, _M16)
            i01 = lax.shift_right_logical(va, 16)
            i10 = jnp.bitwise_and(vb, _M16)
            i11 = lax.shift_right_logical(vb, 16)
            w00v = cb_v[p, 2, pl.ds(s, 16)]
            w01v = cb_v[p, 3, pl.ds(s, 16)]
            w10v = cb_v[p, 4, pl.ds(s, 16)]
            w11v = cb_v[p, 5, pl.ds(s, 16)]
            for tbl_v, st_v in ((tbl0_v, st0_v), (tbl1_v, st1_v)):
                f00a, f00b = _unpack_pair(plsc.load_gather(tbl_v, [i00]))
                f01a, f01b = _unpack_pair(plsc.load_gather(tbl_v, [i01]))
                f10a, f10b = _unpack_pair(plsc.load_gather(tbl_v, [i10]))
                f11a, f11b = _unpack_pair(plsc.load_gather(tbl_v, [i11]))
                ra = ((f00a * w00v + f01a * w01v) + f10a * w10v) + f11a * w11v
                rb = ((f00b * w00v + f01b * w01v) + f10b * w10v) + f11b * w11v
                st_v[p, 0, pl.ds(s, 16)] = ra
                st_v[p, 1, pl.ds(s, 16)] = rb
            return c2

        lax.fori_loop(0, _CHUNK // 16, grp, 0)

    in_h = {0: pltpu.async_copy(cc.at[lvl, 0], cb_v.at[0], sin[0])}
    out_h = [None, None]
    for ci in range(_NCHUNK):
        p = ci % 2
        if ci + 1 < _NCHUNK:
            in_h[ci + 1] = pltpu.async_copy(
                cc.at[lvl, ci + 1], cb_v.at[1 - p], sin[1 - p])
        in_h[ci].wait()
        if out_h[p] is not None:
            out_h[p][0].wait()
            out_h[p][1].wait()
        compute_chunk(p, ci)
        off = ci * _CHUNK
        dst0 = featT.at[b0, pl.ds(lvl * 2, 2), pl.ds(off, _CHUNK)]
        dst1 = featT.at[b0 + 1, pl.ds(lvl * 2, 2), pl.ds(off, _CHUNK)]
        out_h[p] = (pltpu.async_copy(st0_v.at[p], dst0, sout[p][0]),
                    pltpu.async_copy(st1_v.at[p], dst1, sout[p][1]))
    for p in (0, 1):
        if out_h[p] is not None:
            out_h[p][0].wait()
            out_h[p][1].wait()


_TN = 2048


def _mlp_body(x_ref, w1_ref, b1_ref, w2_ref, b2_ref, w3_ref, b3_ref, out_ref):
    x = x_ref[0]
    h = jnp.dot(w1_ref[...], x, preferred_element_type=jnp.float32)
    h = jnp.maximum(h + b1_ref[...], 0.0)
    h = jnp.dot(w2_ref[...], h, preferred_element_type=jnp.float32)
    h = jnp.maximum(h + b2_ref[...], 0.0)
    o = jnp.dot(w3_ref[...], h, preferred_element_type=jnp.float32)
    out_ref[0] = jnp.tanh(o + b3_ref[...])


def _mlp(featT, W1T, b1, W2T, b2, W3T, b3):
    return pl.pallas_call(
        _mlp_body,
        grid=(_BATCH, _N // _TN),
        in_specs=[
            pl.BlockSpec((1, 2 * _TABLE_NUM, _TN), lambda b, i: (b, 0, i)),
            pl.BlockSpec((_HID, 2 * _TABLE_NUM), lambda b, i: (0, 0)),
            pl.BlockSpec((_HID, 1), lambda b, i: (0, 0)),
            pl.BlockSpec((_HID, _HID), lambda b, i: (0, 0)),
            pl.BlockSpec((_HID, 1), lambda b, i: (0, 0)),
            pl.BlockSpec((3, _HID), lambda b, i: (0, 0)),
            pl.BlockSpec((3, 1), lambda b, i: (0, 0)),
        ],
        out_specs=pl.BlockSpec((1, 3, _TN), lambda b, i: (b, 0, i)),
        out_shape=jax.ShapeDtypeStruct((_BATCH, 3, _N), jnp.float32),
    )(featT, W1T, b1.reshape(-1, 1), W2T, b2.reshape(-1, 1),
      W3T, b3.reshape(-1, 1))


def kernel(z, Wg, bg, W1, b1, W2, b2, W3, b3):
    tables = _tablegen(z, Wg, bg).reshape(
        _BATCH, _TABLE_NUM, 2 * _TABLE_SIZE)
    featT = _make_sc_gather()(tables, _CC)
    out = _mlp(featT, W1.T, b1, W2.T, b2, W3.T, b3)
    return out.reshape(_BATCH, 3, _IMG, _IMG)


# speedup vs baseline: 2.2022x; 1.0008x over previous
import functools

import numpy as np
import jax
import jax.numpy as jnp
from jax import lax
from jax.experimental import pallas as pl
from jax.experimental.pallas import tpu as pltpu
from jax.experimental.pallas import tpu_sc as plsc

_TABLE_NUM = 16
_TABLE_SIZE = 8192
_IMG = 256
_N = _IMG * _IMG
_BATCH = 4
_HID = 64
_CHUNK = 4096
_NCHUNK = _N // _CHUNK
_M16 = np.int32(0xFFFF)
_MHI = np.int32(-65536)
_RND = np.int32(0x7FFF)


def _resolutions():
    b = np.exp((np.log(256.0) - np.log(16.0)) / (_TABLE_NUM - 1))
    return np.floor(16.0 * (b ** np.arange(_TABLE_NUM))).astype(np.int64)


def _build_consts():
    res = _resolutions()
    pi2 = np.uint32(2654435761)
    c = (np.arange(_IMG, dtype=np.float32) + np.float32(0.5)) / np.float32(_IMG)
    cc = np.empty((_TABLE_NUM, 6, _N), np.float32)
    for lvl in range(_TABLE_NUM):
        r = np.float32(float(res[lvl]))
        xy = c * r
        x0 = np.floor(xy)
        fr = (xy - x0).astype(np.float32)
        i0 = x0.astype(np.uint32)
        i1 = i0 + np.uint32(1)
        ix0 = i0[:, None]
        ix1 = i1[:, None]
        m0 = (i0 * pi2)[None, :]
        m1 = (i1 * pi2)[None, :]

        def hsh(a, m):
            return (a ^ m) % np.uint32(_TABLE_SIZE)

        hA = (hsh(ix0, m0) | (hsh(ix0, m1) << np.uint32(16))).ravel()
        hB = (hsh(ix1, m0) | (hsh(ix1, m1) << np.uint32(16))).ravel()
        cc[lvl, 0] = hA.view(np.float32)
        cc[lvl, 1] = hB.view(np.float32)
        fx = np.broadcast_to(fr[:, None], (_IMG, _IMG)).astype(np.float32)
        fy = np.broadcast_to(fr[None, :], (_IMG, _IMG)).astype(np.float32)
        gx = np.float32(1.0) - fx
        gy = np.float32(1.0) - fy
        cc[lvl, 2] = (gx * gy).ravel()
        cc[lvl, 3] = (gx * fy).ravel()
        cc[lvl, 4] = (fx * gy).ravel()
        cc[lvl, 5] = (fx * fy).ravel()
    return np.ascontiguousarray(
        cc.reshape(_TABLE_NUM, 6, _NCHUNK, _CHUNK).transpose(0, 2, 1, 3))


_CC = _build_consts()


_CB = 32768


def _tablegen_body(z_ref, wg_ref, bg_ref, out_ref):
    acc = jnp.dot(z_ref[...], wg_ref[...], preferred_element_type=jnp.float32)
    out_ref[...] = jnp.tanh(acc + bg_ref[...])


def _tablegen(z, Wg, bg):
    ncols = Wg.shape[1]
    return pl.pallas_call(
        _tablegen_body,
        grid=(ncols // _CB,),
        in_specs=[
            pl.BlockSpec((_BATCH, 64), lambda i: (0, 0)),
            pl.BlockSpec((64, _CB), lambda i: (0, i)),
            pl.BlockSpec((1, _CB), lambda i: (0, i)),
        ],
        out_specs=pl.BlockSpec((_BATCH, _CB), lambda i: (0, i)),
        out_shape=jax.ShapeDtypeStruct((_BATCH, ncols), jnp.float32),
    )(z, Wg, bg.reshape(1, -1))


_NC = 2


def _pack_table(tblf_v, pk_v):
    iota2 = lax.iota(jnp.int32, 16) * 2

    def body(g, carry):
        ii = iota2 + g * 32
        be = plsc.bitcast(plsc.load_gather(tblf_v, [ii]), jnp.int32)
        bo = plsc.bitcast(plsc.load_gather(tblf_v, [ii + 1]), jnp.int32)
        se = be + jnp.bitwise_and(lax.shift_right_logical(be, 16), 1) + _RND
        so = bo + jnp.bitwise_and(lax.shift_right_logical(bo, 16), 1) + _RND
        word = jnp.bitwise_or(lax.shift_right_logical(se, 16),
                              jnp.bitwise_and(so, _MHI))
        pk_v[pl.ds(g * 16, 16)] = word
        return carry

    lax.fori_loop(0, _TABLE_SIZE // 16, body, 0)


def _unpack_pair(g):
    lo = plsc.bitcast(lax.shift_left(g, 16), jnp.float32)
    hi = plsc.bitcast(jnp.bitwise_and(g, _MHI), jnp.float32)
    return lo, hi


@functools.cache
def _make_sc_gather():
    return functools.partial(
        pl.kernel,
        mesh=plsc.VectorSubcoreMesh(core_axis_name="c", subcore_axis_name="s"),
        compiler_params=pltpu.CompilerParams(needs_layout_passes=False),
        out_type=jax.ShapeDtypeStruct((_BATCH, 2 * _TABLE_NUM, _N), jnp.float32),
        scratch_types=[
            pltpu.VMEM((2 * _TABLE_SIZE,), jnp.float32),
            pltpu.VMEM((_TABLE_SIZE,), jnp.int32),
            pltpu.VMEM((_TABLE_SIZE,), jnp.int32),
            pltpu.VMEM((2, 6, _CHUNK), jnp.float32),
            pltpu.VMEM((2, 2, _CHUNK), jnp.float32),
            pltpu.VMEM((2, 2, _CHUNK), jnp.float32),
            pltpu.SemaphoreType.DMA,
            pltpu.SemaphoreType.DMA,
            pltpu.SemaphoreType.DMA,
            pltpu.SemaphoreType.DMA,
            pltpu.SemaphoreType.DMA,
            pltpu.SemaphoreType.DMA,
        ],
    )(_sc_gather_body)


def _sc_gather_body(tables, cc, featT,
                    tf_v, tbl0_v, tbl1_v, cb_v, st0_v, st1_v,
                    sin0, sin1, so0a, so0b, so1a, so1b):
    wid = lax.axis_index("s") * _NC + lax.axis_index("c")
    lvl = wid % _TABLE_NUM
    b0 = (wid // _TABLE_NUM) * 2
    pltpu.sync_copy(tables.at[b0, lvl], tf_v)
    _pack_table(tf_v, tbl0_v)
    pltpu.sync_copy(tables.at[b0 + 1, lvl], tf_v)
    _pack_table(tf_v, tbl1_v)

    sin = (sin0, sin1)
    sout = ((so0a, so0b), (so1a, so1b))

    def compute_chunk(p, ci):
        def grp(g, c2):
            s = g * 16
            va = plsc.bitcast(cb_v[p, 0, pl.ds(s, 16)], jnp.int32)
            vb = plsc.bitcast(cb_v[p, 1, pl.ds(s, 16)], jnp.int32)
            i00 = jnp.bitwise_and(va, _M16)
            i01 = lax.shift_right_logical(va, 16)
            i10 = jnp.bitwise_and(vb, _M16)
            i11 = lax.shift_right_logical(vb, 16)
            w00v = cb_v[p, 2, pl.ds(s, 16)]
            w01v = cb_v[p, 3, pl.ds(s, 16)]
            w10v = cb_v[p, 4, pl.ds(s, 16)]
            w11v = cb_v[p, 5, pl.ds(s, 16)]
            for tbl_v, st_v in ((tbl0_v, st0_v), (tbl1_v, st1_v)):
                f00a, f00b = _unpack_pair(plsc.load_gather(tbl_v, [i00]))
                f01a, f01b = _unpack_pair(plsc.load_gather(tbl_v, [i01]))
                f10a, f10b = _unpack_pair(plsc.load_gather(tbl_v, [i10]))
                f11a, f11b = _unpack_pair(plsc.load_gather(tbl_v, [i11]))
                ra = ((f00a * w00v + f01a * w01v) + f10a * w10v) + f11a * w11v
                rb = ((f00b * w00v + f01b * w01v) + f10b * w10v) + f11b * w11v
                st_v[p, 0, pl.ds(s, 16)] = ra
                st_v[p, 1, pl.ds(s, 16)] = rb
            return c2

        lax.fori_loop(0, _CHUNK // 16, grp, 0)

    in_h = {0: pltpu.async_copy(cc.at[lvl, 0], cb_v.at[0], sin[0])}
    out_h = [None, None]
    for ci in range(_NCHUNK):
        p = ci % 2
        if ci + 1 < _NCHUNK:
            in_h[ci + 1] = pltpu.async_copy(
                cc.at[lvl, ci + 1], cb_v.at[1 - p], sin[1 - p])
        in_h[ci].wait()
        if out_h[p] is not None:
            out_h[p][0].wait()
            out_h[p][1].wait()
        compute_chunk(p, ci)
        off = ci * _CHUNK
        dst0 = featT.at[b0, pl.ds(lvl * 2, 2), pl.ds(off, _CHUNK)]
        dst1 = featT.at[b0 + 1, pl.ds(lvl * 2, 2), pl.ds(off, _CHUNK)]
        out_h[p] = (pltpu.async_copy(st0_v.at[p], dst0, sout[p][0]),
                    pltpu.async_copy(st1_v.at[p], dst1, sout[p][1]))
    for p in (0, 1):
        if out_h[p] is not None:
            out_h[p][0].wait()
            out_h[p][1].wait()


_TN = 2048


def _mlp_body(x_ref, w1_ref, b1_ref, w2_ref, b2_ref, w3_ref, b3_ref, out_ref):
    x = x_ref[0]
    h = jnp.dot(w1_ref[...], x, preferred_element_type=jnp.float32)
    h = jnp.maximum(h + b1_ref[...], 0.0)
    h = jnp.dot(w2_ref[...], h, preferred_element_type=jnp.float32)
    h = jnp.maximum(h + b2_ref[...], 0.0)
    o = jnp.dot(w3_ref[...], h, preferred_element_type=jnp.float32)
    out_ref[0] = jnp.tanh(o + b3_ref[...])


def _mlp(featT, W1T, b1, W2T, b2, W3T, b3):
    return pl.pallas_call(
        _mlp_body,
        grid=(_BATCH, _N // _TN),
        in_specs=[
            pl.BlockSpec((1, 2 * _TABLE_NUM, _TN), lambda b, i: (b, 0, i)),
            pl.BlockSpec((_HID, 2 * _TABLE_NUM), lambda b, i: (0, 0)),
            pl.BlockSpec((_HID, 1), lambda b, i: (0, 0)),
            pl.BlockSpec((_HID, _HID), lambda b, i: (0, 0)),
            pl.BlockSpec((_HID, 1), lambda b, i: (0, 0)),
            pl.BlockSpec((3, _HID), lambda b, i: (0, 0)),
            pl.BlockSpec((3, 1), lambda b, i: (0, 0)),
        ],
        out_specs=pl.BlockSpec((1, 3, _TN), lambda b, i: (b, 0, i)),
        out_shape=jax.ShapeDtypeStruct((_BATCH, 3, _N), jnp.float32),
    )(featT, W1T, b1.reshape(-1, 1), W2T, b2.reshape(-1, 1),
      W3T, b3.reshape(-1, 1))


def kernel(z, Wg, bg, W1, b1, W2, b2, W3, b3):
    tables = _tablegen(z, Wg, bg).reshape(
        _BATCH, _TABLE_NUM, 2 * _TABLE_SIZE)
    featT = _make_sc_gather()(tables, _CC)
    out = _mlp(featT, W1.T, b1, W2.T, b2, W3.T, b3)
    return out.reshape(_BATCH, 3, _IMG, _IMG)


# speedup vs baseline: 2.9244x; 1.3280x over previous
import functools

import numpy as np
import jax
import jax.numpy as jnp
from jax import lax
from jax.experimental import pallas as pl
from jax.experimental.pallas import tpu as pltpu
from jax.experimental.pallas import tpu_sc as plsc

_TABLE_NUM = 16
_TABLE_SIZE = 8192
_IMG = 256
_N = _IMG * _IMG
_BATCH = 4
_HID = 64
_CHUNK = 4096
_NCHUNK = _N // _CHUNK
_M16 = np.int32(0xFFFF)
_MHI = np.int32(-65536)
_RND = np.int32(0x7FFF)


def _resolutions():
    b = np.exp((np.log(256.0) - np.log(16.0)) / (_TABLE_NUM - 1))
    return np.floor(16.0 * (b ** np.arange(_TABLE_NUM))).astype(np.int64)


def _build_consts():
    res = _resolutions()
    pi2 = np.uint32(2654435761)
    c = (np.arange(_IMG, dtype=np.float32) + np.float32(0.5)) / np.float32(_IMG)
    cc = np.empty((_TABLE_NUM, 6, _N), np.float32)
    for lvl in range(_TABLE_NUM):
        r = np.float32(float(res[lvl]))
        xy = c * r
        x0 = np.floor(xy)
        fr = (xy - x0).astype(np.float32)
        i0 = x0.astype(np.uint32)
        i1 = i0 + np.uint32(1)
        ix0 = i0[:, None]
        ix1 = i1[:, None]
        m0 = (i0 * pi2)[None, :]
        m1 = (i1 * pi2)[None, :]

        def hsh(a, m):
            return (a ^ m) % np.uint32(_TABLE_SIZE)

        hA = (hsh(ix0, m0) | (hsh(ix0, m1) << np.uint32(16))).ravel()
        hB = (hsh(ix1, m0) | (hsh(ix1, m1) << np.uint32(16))).ravel()
        cc[lvl, 0] = hA.view(np.float32)
        cc[lvl, 1] = hB.view(np.float32)
        fx = np.broadcast_to(fr[:, None], (_IMG, _IMG)).astype(np.float32)
        fy = np.broadcast_to(fr[None, :], (_IMG, _IMG)).astype(np.float32)
        gx = np.float32(1.0) - fx
        gy = np.float32(1.0) - fy
        cc[lvl, 2] = (gx * gy).ravel()
        cc[lvl, 3] = (gx * fy).ravel()
        cc[lvl, 4] = (fx * gy).ravel()
        cc[lvl, 5] = (fx * fy).ravel()
    return np.ascontiguousarray(
        cc.reshape(_TABLE_NUM, 6, _NCHUNK, _CHUNK).transpose(0, 2, 1, 3))


_CC = _build_consts()


_CB = 32768


def _tablegen_body(z_ref, wg_ref, bg_ref, out_ref):
    acc = jnp.dot(z_ref[...], wg_ref[...], preferred_element_type=jnp.float32)
    out_ref[...] = jnp.tanh(acc + bg_ref[...])


def _tablegen(z, Wg, bg):
    ncols = Wg.shape[1]
    return pl.pallas_call(
        _tablegen_body,
        grid=(ncols // _CB,),
        in_specs=[
            pl.BlockSpec((_BATCH, 64), lambda i: (0, 0)),
            pl.BlockSpec((64, _CB), lambda i: (0, i)),
            pl.BlockSpec((1, _CB), lambda i: (0, i)),
        ],
        out_specs=pl.BlockSpec((_BATCH, _CB), lambda i: (0, i)),
        out_shape=jax.ShapeDtypeStruct((_BATCH, ncols), jnp.float32),
    )(z, Wg, bg.reshape(1, -1))


_NC = 2


def _pack_table(tblf_v, pk_v):
    iota2 = lax.iota(jnp.int32, 16) * 2

    def body(g):
        ii = iota2 + g * 32
        be = plsc.bitcast(plsc.load_gather(tblf_v, [ii]), jnp.int32)
        bo = plsc.bitcast(plsc.load_gather(tblf_v, [ii + 1]), jnp.int32)
        se = be + jnp.bitwise_and(lax.shift_right_logical(be, 16), 1) + _RND
        so = bo + jnp.bitwise_and(lax.shift_right_logical(bo, 16), 1) + _RND
        word = jnp.bitwise_or(lax.shift_right_logical(se, 16),
                              jnp.bitwise_and(so, _MHI))
        pk_v[pl.ds(g * 16, 16)] = word

    plsc.parallel_loop(0, _TABLE_SIZE // 16, unroll=4)(body)


def _unpack_pair(g):
    lo = plsc.bitcast(lax.shift_left(g, 16), jnp.float32)
    hi = plsc.bitcast(jnp.bitwise_and(g, _MHI), jnp.float32)
    return lo, hi


@functools.cache
def _make_sc_gather():
    return functools.partial(
        pl.kernel,
        mesh=plsc.VectorSubcoreMesh(core_axis_name="c", subcore_axis_name="s"),
        compiler_params=pltpu.CompilerParams(needs_layout_passes=False),
        out_type=jax.ShapeDtypeStruct((_BATCH, 2 * _TABLE_NUM, _N), jnp.float32),
        scratch_types=[
            pltpu.VMEM((2 * _TABLE_SIZE,), jnp.float32),
            pltpu.VMEM((_TABLE_SIZE,), jnp.int32),
            pltpu.VMEM((_TABLE_SIZE,), jnp.int32),
            pltpu.VMEM((2, 6, _CHUNK), jnp.float32),
            pltpu.VMEM((2, 2, _CHUNK), jnp.float32),
            pltpu.VMEM((2, 2, _CHUNK), jnp.float32),
            pltpu.SemaphoreType.DMA,
            pltpu.SemaphoreType.DMA,
            pltpu.SemaphoreType.DMA,
            pltpu.SemaphoreType.DMA,
            pltpu.SemaphoreType.DMA,
            pltpu.SemaphoreType.DMA,
        ],
    )(_sc_gather_body)


def _sc_gather_body(tables, cc, featT,
                    tf_v, tbl0_v, tbl1_v, cb_v, st0_v, st1_v,
                    sin0, sin1, so0a, so0b, so1a, so1b):
    wid = lax.axis_index("s") * _NC + lax.axis_index("c")
    lvl = wid % _TABLE_NUM
    b0 = (wid // _TABLE_NUM) * 2
    pltpu.sync_copy(tables.at[b0, lvl], tf_v)
    _pack_table(tf_v, tbl0_v)
    pltpu.sync_copy(tables.at[b0 + 1, lvl], tf_v)
    _pack_table(tf_v, tbl1_v)

    sin = (sin0, sin1)
    sout = ((so0a, so0b), (so1a, so1b))

    def compute_chunk(p, ci):
        def grp(g):
            s = g * 16
            va = plsc.bitcast(cb_v[p, 0, pl.ds(s, 16)], jnp.int32)
            vb = plsc.bitcast(cb_v[p, 1, pl.ds(s, 16)], jnp.int32)
            i00 = jnp.bitwise_and(va, _M16)
            i01 = lax.shift_right_logical(va, 16)
            i10 = jnp.bitwise_and(vb, _M16)
            i11 = lax.shift_right_logical(vb, 16)
            w00v = cb_v[p, 2, pl.ds(s, 16)]
            w01v = cb_v[p, 3, pl.ds(s, 16)]
            w10v = cb_v[p, 4, pl.ds(s, 16)]
            w11v = cb_v[p, 5, pl.ds(s, 16)]
            for tbl_v, st_v in ((tbl0_v, st0_v), (tbl1_v, st1_v)):
                f00a, f00b = _unpack_pair(plsc.load_gather(tbl_v, [i00]))
                f01a, f01b = _unpack_pair(plsc.load_gather(tbl_v, [i01]))
                f10a, f10b = _unpack_pair(plsc.load_gather(tbl_v, [i10]))
                f11a, f11b = _unpack_pair(plsc.load_gather(tbl_v, [i11]))
                ra = ((f00a * w00v + f01a * w01v) + f10a * w10v) + f11a * w11v
                rb = ((f00b * w00v + f01b * w01v) + f10b * w10v) + f11b * w11v
                st_v[p, 0, pl.ds(s, 16)] = ra
                st_v[p, 1, pl.ds(s, 16)] = rb

        plsc.parallel_loop(0, _CHUNK // 16, unroll=2)(grp)

    in_h = {0: pltpu.async_copy(cc.at[lvl, 0], cb_v.at[0], sin[0])}
    out_h = [None, None]
    for ci in range(_NCHUNK):
        p = ci % 2
        if ci + 1 < _NCHUNK:
            in_h[ci + 1] = pltpu.async_copy(
                cc.at[lvl, ci + 1], cb_v.at[1 - p], sin[1 - p])
        in_h[ci].wait()
        if out_h[p] is not None:
            out_h[p][0].wait()
            out_h[p][1].wait()
        compute_chunk(p, ci)
        off = ci * _CHUNK
        dst0 = featT.at[b0, pl.ds(lvl * 2, 2), pl.ds(off, _CHUNK)]
        dst1 = featT.at[b0 + 1, pl.ds(lvl * 2, 2), pl.ds(off, _CHUNK)]
        out_h[p] = (pltpu.async_copy(st0_v.at[p], dst0, sout[p][0]),
                    pltpu.async_copy(st1_v.at[p], dst1, sout[p][1]))
    for p in (0, 1):
        if out_h[p] is not None:
            out_h[p][0].wait()
            out_h[p][1].wait()


_TN = 2048


def _mlp_body(x_ref, w1_ref, b1_ref, w2_ref, b2_ref, w3_ref, b3_ref, out_ref):
    x = x_ref[0]
    h = jnp.dot(w1_ref[...], x, preferred_element_type=jnp.float32)
    h = jnp.maximum(h + b1_ref[...], 0.0)
    h = jnp.dot(w2_ref[...], h, preferred_element_type=jnp.float32)
    h = jnp.maximum(h + b2_ref[...], 0.0)
    o = jnp.dot(w3_ref[...], h, preferred_element_type=jnp.float32)
    out_ref[0] = jnp.tanh(o + b3_ref[...])


def _mlp(featT, W1T, b1, W2T, b2, W3T, b3):
    return pl.pallas_call(
        _mlp_body,
        grid=(_BATCH, _N // _TN),
        in_specs=[
            pl.BlockSpec((1, 2 * _TABLE_NUM, _TN), lambda b, i: (b, 0, i)),
            pl.BlockSpec((_HID, 2 * _TABLE_NUM), lambda b, i: (0, 0)),
            pl.BlockSpec((_HID, 1), lambda b, i: (0, 0)),
            pl.BlockSpec((_HID, _HID), lambda b, i: (0, 0)),
            pl.BlockSpec((_HID, 1), lambda b, i: (0, 0)),
            pl.BlockSpec((3, _HID), lambda b, i: (0, 0)),
            pl.BlockSpec((3, 1), lambda b, i: (0, 0)),
        ],
        out_specs=pl.BlockSpec((1, 3, _TN), lambda b, i: (b, 0, i)),
        out_shape=jax.ShapeDtypeStruct((_BATCH, 3, _N), jnp.float32),
    )(featT, W1T, b1.reshape(-1, 1), W2T, b2.reshape(-1, 1),
      W3T, b3.reshape(-1, 1))


def kernel(z, Wg, bg, W1, b1, W2, b2, W3, b3):
    tables = _tablegen(z, Wg, bg).reshape(
        _BATCH, _TABLE_NUM, 2 * _TABLE_SIZE)
    featT = _make_sc_gather()(tables, _CC)
    out = _mlp(featT, W1.T, b1, W2.T, b2, W3.T, b3)
    return out.reshape(_BATCH, 3, _IMG, _IMG)
